# baseline (speedup 1.0000x reference)
import functools

import jax
import jax.numpy as jnp
from jax import lax
from jax.experimental import pallas as pl
from jax.experimental.pallas import tpu as pltpu
from jax.experimental.pallas import tpu_sc as plsc

_D = 64
_DP = 128
_NW = 32
_CHUNK = 128
_GROUP = 2
_NBUF = 2


@functools.partial(jax.jit, static_argnames=("total",))
def _sc_gather(idx3d, table_p, total):
    b_per_w = total // _NW
    n_chunks = b_per_w // _CHUNK
    n_rounds = n_chunks // _GROUP
    assert n_rounds % _NBUF == 0
    rows_per_buf = _GROUP * _CHUNK
    mesh = plsc.VectorSubcoreMesh(core_axis_name="c", subcore_axis_name="s")

    @functools.partial(
        pl.kernel,
        out_type=jax.ShapeDtypeStruct((total, _DP), jnp.float32),
        mesh=mesh,
        scratch_types=[
            pltpu.VMEM((n_chunks, _CHUNK), jnp.int32),
            pltpu.VMEM((_NBUF, rows_per_buf, _DP), jnp.float32),
            pltpu.SemaphoreType.DMA((_NBUF,)),
            pltpu.SemaphoreType.DMA((_NBUF,)),
        ],
    )
    def gather_kernel(idx_hbm, table_hbm, out_hbm, idx_v, rows_v, gsem, osem):
        cid = lax.axis_index("c")
        sid = lax.axis_index("s")
        wid = sid * 2 + cid
        base = wid * b_per_w

        pltpu.sync_copy(idx_hbm.at[wid], idx_v)

        def fire_gathers(rd, s):
            for g in range(_GROUP):
                pltpu.async_copy(
                    table_hbm.at[idx_v.at[rd * _GROUP + g]],
                    rows_v.at[s].at[pl.ds(g * _CHUNK, _CHUNK)],
                    gsem.at[s])

        def drain_gathers(s):
            pltpu.make_async_copy(table_hbm.at[pl.ds(0, rows_per_buf)],
                                  rows_v.at[s], gsem.at[s]).wait()

        def start_out(rd, s):
            pltpu.async_copy(
                rows_v.at[s],
                out_hbm.at[pl.ds(base + rd * rows_per_buf, rows_per_buf)],
                osem.at[s])

        def wait_out(s):
            pltpu.make_async_copy(rows_v.at[s],
                                  out_hbm.at[pl.ds(base, rows_per_buf)],
                                  osem.at[s]).wait()

        for s in range(_NBUF):
            fire_gathers(s, s)

        @pl.loop(0, n_rounds - _NBUF, step=_NBUF)
        def _body(r):
            for s in range(_NBUF):
                rd = r + s
                drain_gathers(s)
                start_out(rd, s)
                wait_out(s)
                fire_gathers(rd + _NBUF, s)

        for s in range(_NBUF):
            drain_gathers(s)
            start_out(n_rounds - _NBUF + s, s)
            wait_out(s)

    return gather_kernel(idx3d, table_p)


def kernel(words, table):
    batch, hist = words.shape
    hist_p = (hist + 7) // 8 * 8
    total = batch * hist_p
    idx = jnp.pad(words.astype(jnp.int32), ((0, 0), (0, hist_p - hist)))
    idx3d = idx.reshape(_NW, total // (_NW * _CHUNK), _CHUNK)
    table_p = jnp.pad(table, ((0, 0), (0, _DP - _D)))
    out = _sc_gather(idx3d, table_p, total)
    return out[:, :_D].reshape(batch, hist_p, _D)[:, :hist, :]


# speedup vs baseline: 5.1327x; 5.1327x over previous
import functools

import jax
import jax.numpy as jnp
from jax import lax
from jax.experimental import pallas as pl
from jax.experimental.pallas import tpu as pltpu
from jax.experimental.pallas import tpu_sc as plsc

_D = 64
_DP = 128
_NW = 32
_CHUNK = 128
_GROUP = 2
_NBUF = 2


@functools.partial(jax.jit, static_argnames=("total",))
def _sc_gather(idx3d, table_p, total):
    b_per_w = total // _NW
    n_chunks = b_per_w // _CHUNK
    n_rounds = n_chunks // _GROUP
    assert n_rounds % _NBUF == 0
    rows_per_buf = _GROUP * _CHUNK
    mesh = plsc.VectorSubcoreMesh(core_axis_name="c", subcore_axis_name="s")

    @functools.partial(
        pl.kernel,
        out_type=jax.ShapeDtypeStruct((total, _DP), jnp.float32),
        mesh=mesh,
        scratch_types=[
            pltpu.VMEM((n_chunks, _CHUNK), jnp.int32),
            pltpu.VMEM((_NBUF, rows_per_buf, _DP), jnp.float32),
            pltpu.SemaphoreType.DMA((_NBUF,)),
            pltpu.SemaphoreType.DMA((_NBUF,)),
        ],
    )
    def gather_kernel(idx_hbm, table_hbm, out_hbm, idx_v, rows_v, gsem, osem):
        cid = lax.axis_index("c")
        sid = lax.axis_index("s")
        wid = sid * 2 + cid
        base = wid * b_per_w

        pltpu.sync_copy(idx_hbm.at[wid], idx_v)

        def fire_gathers(rd, s):
            for g in range(_GROUP):
                pltpu.async_copy(
                    table_hbm.at[idx_v.at[rd * _GROUP + g]],
                    rows_v.at[s].at[pl.ds(g * _CHUNK, _CHUNK)],
                    gsem.at[s])

        def drain_gathers(s):
            pltpu.make_async_copy(table_hbm.at[pl.ds(0, rows_per_buf)],
                                  rows_v.at[s], gsem.at[s]).wait()

        def start_out(rd, s):
            pltpu.async_copy(
                rows_v.at[s],
                out_hbm.at[pl.ds(base + rd * rows_per_buf, rows_per_buf)],
                osem.at[s])

        def wait_out(s):
            pltpu.make_async_copy(rows_v.at[s],
                                  out_hbm.at[pl.ds(base, rows_per_buf)],
                                  osem.at[s]).wait()

        for s in range(_NBUF):
            fire_gathers(s, s)

        @pl.loop(0, n_rounds - _NBUF, step=_NBUF)
        def _body(r):
            for s in range(_NBUF):
                rd = r + s
                drain_gathers(s)
                start_out(rd, s)
                wait_out(s)
                fire_gathers(rd + _NBUF, s)

        for s in range(_NBUF):
            drain_gathers(s)
            start_out(n_rounds - _NBUF + s, s)
            wait_out(s)

    return gather_kernel(idx3d, table_p)


def kernel(words, table):
    batch, hist = words.shape
    hist_p = (hist + 7) // 8 * 8
    total = batch * hist_p
    w32 = words.astype(jnp.int32)
    idx = jnp.concatenate([w32, w32[:, : hist_p - hist]], axis=1)
    idx3d = idx.reshape(_NW, total // (_NW * _CHUNK), _CHUNK)
    table_p = jnp.pad(table, ((0, 0), (0, _DP - _D)))
    out = _sc_gather(idx3d, table_p, total)
    return out[:, :_D].reshape(batch, hist_p, _D)[:, :hist, :]
